# baseline (speedup 1.0000x reference)
import functools

import jax
import jax.numpy as jnp
from jax import lax
from jax.experimental import pallas as pl
from jax.experimental.pallas import tpu as pltpu
from jax.experimental.pallas import tpu_sc as plsc

N_ROWS = 131072
N_COLS = 128

_info = plsc.get_sparse_core_info()
NC, NS, L = _info.num_cores, _info.num_subcores, _info.num_lanes
NW = NC * NS
ROWS_PER_W = N_ROWS // NW
CHUNK_ROWS = 320
_CHUNK_SIZES = [CHUNK_ROWS] * (ROWS_PER_W // CHUNK_ROWS)
if ROWS_PER_W % CHUNK_ROWS:
    _CHUNK_SIZES.append(ROWS_PER_W % CHUNK_ROWS)
_CHUNK_OFFS = [0]
for _s in _CHUNK_SIZES[:-1]:
    _CHUNK_OFFS.append(_CHUNK_OFFS[-1] + _s)
N_CHUNKS = len(_CHUNK_SIZES)
CHUNK_ELEMS = CHUNK_ROWS * N_COLS
G = N_COLS // L
NBUF = 3

_mesh = plsc.VectorSubcoreMesh(core_axis_name="c", subcore_axis_name="s")


@functools.partial(
    pl.kernel,
    mesh=_mesh,
    out_type=jax.ShapeDtypeStruct((N_ROWS * N_COLS,), jnp.float32),
    scratch_types=[
        pltpu.VMEM((N_COLS,), jnp.int32),
        [pltpu.VMEM((CHUNK_ELEMS,), jnp.float32) for _ in range(NBUF)],
        [pltpu.SemaphoreType.DMA for _ in range(NBUF)],
        [pltpu.SemaphoreType.DMA for _ in range(NBUF)],
        pltpu.SemaphoreType.DMA,
    ],
    compiler_params=pltpu.CompilerParams(
        needs_layout_passes=False,
        disable_bounds_checks=True,
        disable_semaphore_checks=True,
        skip_device_barrier=True,
    ),
)
def _shuffle(x_hbm, idx_hbm, out_hbm, idx_v, buf_v, in_sem, out_sem, idx_sem):
    wid = lax.axis_index("s") * NC + lax.axis_index("c")
    idx_d = pltpu.async_copy(idx_hbm, idx_v, idx_sem)
    base_w = wid * (ROWS_PER_W * N_COLS)

    def in_copy(ci, b):
        elems = _CHUNK_SIZES[ci] * N_COLS
        return pltpu.async_copy(
            x_hbm.at[pl.ds(base_w + _CHUNK_OFFS[ci] * N_COLS, elems)],
            buf_v[b].at[pl.ds(0, elems)], in_sem[b])

    def permute_chunk(buf, rows):
        @plsc.parallel_loop(0, rows, step=1, unroll=4)
        def row_body(r):
            rb = r * N_COLS
            vs = [plsc.load_gather(buf, [col_idx[g] + rb]) for g in range(G)]
            for g in range(G):
                buf[pl.ds(rb + g * L, L)] = vs[g]

    in_d = [None] * NBUF
    out_d = [None] * NBUF
    in_d[0] = in_copy(0, 0)
    in_d[1] = in_copy(1, 1)
    idx_d.wait()
    col_idx = [idx_v[pl.ds(g * L, L)] for g in range(G)]
    for ci in range(N_CHUNKS):
        b = ci % NBUF
        in_d[b].wait()
        permute_chunk(buf_v[b], _CHUNK_SIZES[ci])
        elems = _CHUNK_SIZES[ci] * N_COLS
        out_d[b] = pltpu.async_copy(
            buf_v[b].at[pl.ds(0, elems)],
            out_hbm.at[pl.ds(base_w + _CHUNK_OFFS[ci] * N_COLS, elems)],
            out_sem[b])
        if ci + 2 < N_CHUNKS:
            nb = (ci + 2) % NBUF
            if out_d[nb] is not None:
                out_d[nb].wait()
            in_d[nb] = in_copy(ci + 2, nb)
    for b in range(NBUF):
        if out_d[b] is not None:
            out_d[b].wait()


def kernel(input, shuffle_indices):
    out_flat = _shuffle(input.reshape(-1), shuffle_indices)
    return out_flat.reshape(N_ROWS, N_COLS)
